# baseline (speedup 1.0000x reference)
import functools

import jax
import jax.numpy as jnp
from jax import lax
from jax.experimental import pallas as pl
from jax.experimental.pallas import tpu as pltpu
from jax.experimental.pallas import tpu_sc as plsc

V, E, NCLS, B, S = 1000, 64, 10, 4096, 200
NC, NS, L = 2, 16, 16
NW = NC * NS
BPW = B // NW
NFULL = S // L
NREM = S - NFULL * L
EW = E // 4
OW = E // 2
CPAD = 128
BMASK = 0x00FF00FF

_PERM = ([4 * (s // 2) + 2 * (s % 2) for s in range(32)]
         + [4 * (s // 2) + 2 * (s % 2) + 1 for s in range(32)])

_mesh = plsc.VectorSubcoreMesh(
    core_axis_name="c", subcore_axis_name="s", num_cores=2)


def _bcast_lane(vec, j):
    return lax.gather(
        vec,
        jnp.full((L, 1), j, jnp.int32),
        lax.GatherDimensionNumbers(
            offset_dims=(), collapsed_slice_dims=(0,), start_index_map=(0,)),
        (1,),
        mode=lax.GatherScatterMode.PROMISE_IN_BOUNDS,
    )


@functools.partial(
    pl.kernel,
    out_type=jax.ShapeDtypeStruct((B * OW,), jnp.int32),
    mesh=_mesh,
    scratch_types=[
        pltpu.VMEM((BPW * S + L - NREM,), jnp.int32),
        pltpu.VMEM((V * EW,), jnp.int32),
        pltpu.VMEM((BPW * OW,), jnp.int32),
    ],
    compiler_params=pltpu.CompilerParams(needs_layout_passes=False),
)
def _pool_sc(idx_hbm, tbl_hbm, out_hbm, idx_v, tbl_v, out_v):
    wid = lax.axis_index("s") * NC + lax.axis_index("c")
    base = wid * BPW
    pltpu.sync_copy(tbl_hbm, tbl_v)
    pltpu.sync_copy(idx_hbm.at[pl.ds(base * S, BPW * S)],
                    idx_v.at[pl.ds(0, BPW * S)])
    lanes = lax.iota(jnp.int32, L)
    zero = jnp.zeros((2 * L,), jnp.uint16)

    def gather_max(idxv, j, ae, ao):
        addr = _bcast_lane(idxv, j) * EW + lanes
        w = plsc.load_gather(tbl_v, [addr])
        ev = plsc.bitcast(w & BMASK, jnp.uint16)
        od = plsc.bitcast(lax.shift_right_logical(w, 8) & BMASK, jnp.uint16)
        return jnp.maximum(ae, ev), jnp.maximum(ao, od)

    def row_body(row, _):
        def chunk_body(c, accs):
            idxv = idx_v[pl.ds(row * S + c * L, L)]
            ae, ao = accs
            for j in range(L):
                ae, ao = gather_max(idxv, j, ae, ao)
            return (ae, ao)

        ae, ao = lax.fori_loop(0, NFULL, chunk_body, (zero, zero))
        idxv = idx_v[pl.ds(row * S + NFULL * L, L)]
        for j in range(NREM):
            ae, ao = gather_max(idxv, j, ae, ao)
        out_v[pl.ds(row * OW, L)] = plsc.bitcast(ae, jnp.int32)
        out_v[pl.ds(row * OW + L, L)] = plsc.bitcast(ao, jnp.int32)
        return 0

    lax.fori_loop(0, BPW, row_body, 0)
    pltpu.sync_copy(out_v, out_hbm.at[pl.ds(base * OW, BPW * OW)])


BLK = 512


def _dense_tc(x_ref, w_ref, b_ref, o_ref):
    logits = jnp.dot(x_ref[...], w_ref[...],
                     preferred_element_type=jnp.float32) + b_ref[...]
    m = jnp.max(logits, axis=1, keepdims=True)
    e = jnp.exp(logits - m)
    o_ref[...] = e / jnp.sum(e, axis=1, keepdims=True)


_dense_call = pl.pallas_call(
    _dense_tc,
    grid=(B // BLK,),
    in_specs=[
        pl.BlockSpec((BLK, E), lambda i: (i, 0)),
        pl.BlockSpec((E, CPAD), lambda i: (0, 0)),
        pl.BlockSpec((1, CPAD), lambda i: (0, 0)),
    ],
    out_specs=pl.BlockSpec((BLK, CPAD), lambda i: (i, 0)),
    out_shape=jax.ShapeDtypeStruct((B, CPAD), jnp.float32),
)


def kernel(indices, table, W, b):
    amax = jnp.maximum(jnp.max(jnp.abs(table)), 1e-30)
    scale = amax / 127.0
    q = (jnp.round(table * (127.0 / amax)).clip(-127, 127)
         .astype(jnp.int16) + 128).astype(jnp.uint8)
    tbl_p = lax.bitcast_convert_type(
        q.reshape(V, EW, 4), jnp.int32).reshape(-1)
    pooled = _pool_sc(indices.reshape(-1), tbl_p)
    x = lax.bitcast_convert_type(
        pooled.reshape(B, OW), jnp.uint16).reshape(B, E).astype(jnp.bfloat16)
    w_eff = (W * scale)[jnp.array(_PERM), :].astype(jnp.bfloat16)
    w_p = jnp.pad(w_eff, ((0, 0), (0, CPAD - NCLS)))
    b_eff = b - 128.0 * scale * jnp.sum(W, axis=0)
    b_p = jnp.concatenate(
        [b_eff, jnp.full((CPAD - NCLS,), -1e30, jnp.float32)]).reshape(1, CPAD)
    return _dense_call(x, w_p, b_p)[:, :NCLS]


# speedup vs baseline: 1.1035x; 1.1035x over previous
import functools

import jax
import jax.numpy as jnp
from jax import lax
from jax.experimental import pallas as pl
from jax.experimental.pallas import tpu as pltpu
from jax.experimental.pallas import tpu_sc as plsc

V, E, NCLS, B, S = 1000, 64, 10, 4096, 200
NC, NS, L = 2, 16, 16
NW = NC * NS
BPW = B // NW
NFULL = S // L
NREM = S - NFULL * L
EW = L
OW = E // 2
CPAD = 128

_mesh = plsc.VectorSubcoreMesh(
    core_axis_name="c", subcore_axis_name="s", num_cores=2)


def _bcast_lane(vec, j):
    return lax.gather(
        vec,
        jnp.full((L, 1), j, jnp.int32),
        lax.GatherDimensionNumbers(
            offset_dims=(), collapsed_slice_dims=(0,), start_index_map=(0,)),
        (1,),
        mode=lax.GatherScatterMode.PROMISE_IN_BOUNDS,
    )


@functools.partial(
    pl.kernel,
    out_type=jax.ShapeDtypeStruct((B * OW,), jnp.int32),
    mesh=_mesh,
    scratch_types=[
        pltpu.VMEM((BPW * S + L - NREM,), jnp.int32),
        pltpu.VMEM((V * EW,), jnp.int32),
        pltpu.VMEM((V * EW,), jnp.int32),
        pltpu.VMEM((BPW * OW,), jnp.int32),
    ],
    compiler_params=pltpu.CompilerParams(needs_layout_passes=False),
)
def _pool_sc(idx_hbm, ta_hbm, tb_hbm, out_hbm, idx_v, ta_v, tb_v, out_v):
    wid = lax.axis_index("s") * NC + lax.axis_index("c")
    base = wid * BPW
    pltpu.sync_copy(ta_hbm, ta_v)
    pltpu.sync_copy(tb_hbm, tb_v)
    pltpu.sync_copy(idx_hbm.at[pl.ds(base * S, BPW * S)],
                    idx_v.at[pl.ds(0, BPW * S)])
    lanes = lax.iota(jnp.int32, L)
    ninf = jnp.full((2 * L,), -jnp.inf, jnp.bfloat16)

    def gather_max(idxs, j, aa, ab):
        addr = _bcast_lane(idxs, j) + lanes
        wa = plsc.bitcast(plsc.load_gather(ta_v, [addr]), jnp.bfloat16)
        wb = plsc.bitcast(plsc.load_gather(tb_v, [addr]), jnp.bfloat16)
        return jnp.maximum(aa, wa), jnp.maximum(ab, wb)

    def row_body(row, _):
        def chunk_body(c, accs):
            idxs = idx_v[pl.ds(row * S + c * L, L)] * EW
            a0, b0, a1, b1 = accs
            for j in range(0, L, 2):
                a0, b0 = gather_max(idxs, j, a0, b0)
                a1, b1 = gather_max(idxs, j + 1, a1, b1)
            return (a0, b0, a1, b1)

        a0, b0, a1, b1 = lax.fori_loop(
            0, NFULL, chunk_body, (ninf, ninf, ninf, ninf))
        idxs = idx_v[pl.ds(row * S + NFULL * L, L)] * EW
        for j in range(0, NREM, 2):
            a0, b0 = gather_max(idxs, j, a0, b0)
            a1, b1 = gather_max(idxs, j + 1, a1, b1)
        aa, ab = jnp.maximum(a0, a1), jnp.maximum(b0, b1)
        out_v[pl.ds(row * OW, L)] = plsc.bitcast(aa, jnp.int32)
        out_v[pl.ds(row * OW + L, L)] = plsc.bitcast(ab, jnp.int32)
        return 0

    lax.fori_loop(0, BPW, row_body, 0)
    pltpu.sync_copy(out_v, out_hbm.at[pl.ds(base * OW, BPW * OW)])


BLK = 512


def _dense_tc(x_ref, we_ref, wo_ref, b_ref, o_ref):
    w = x_ref[...]
    xe = lax.bitcast_convert_type(w << 16, jnp.float32)
    xo = lax.bitcast_convert_type(
        w & jnp.int32(-65536), jnp.float32)
    logits = (jnp.dot(xe, we_ref[...], preferred_element_type=jnp.float32)
              + jnp.dot(xo, wo_ref[...], preferred_element_type=jnp.float32)
              + b_ref[...])
    m = jnp.max(logits, axis=1, keepdims=True)
    e = jnp.exp(logits - m)
    o_ref[...] = e / jnp.sum(e, axis=1, keepdims=True)


_dense_call = pl.pallas_call(
    _dense_tc,
    grid=(B // BLK,),
    in_specs=[
        pl.BlockSpec((BLK, OW), lambda i: (i, 0)),
        pl.BlockSpec((OW, CPAD), lambda i: (0, 0)),
        pl.BlockSpec((OW, CPAD), lambda i: (0, 0)),
        pl.BlockSpec((1, CPAD), lambda i: (0, 0)),
    ],
    out_specs=pl.BlockSpec((BLK, CPAD), lambda i: (i, 0)),
    out_shape=jax.ShapeDtypeStruct((B, CPAD), jnp.float32),
)


def kernel(indices, table, W, b):
    tp = lax.bitcast_convert_type(
        table.astype(jnp.bfloat16).reshape(V, 2, EW, 2), jnp.int32)
    ta = tp[:, 0].reshape(-1)
    tb = tp[:, 1].reshape(-1)
    pooled = _pool_sc(indices.reshape(-1), ta, tb)
    w_e = jnp.pad(W[0::2], ((0, 0), (0, CPAD - NCLS)))
    w_o = jnp.pad(W[1::2], ((0, 0), (0, CPAD - NCLS)))
    b_p = jnp.concatenate(
        [b, jnp.full((CPAD - NCLS,), -1e30, jnp.float32)]).reshape(1, CPAD)
    return _dense_call(pooled.reshape(B, OW), w_e, w_o, b_p)[:, :NCLS]
